# baseline (speedup 1.0000x reference)
import jax
import jax.numpy as jnp
from jax.experimental import pallas as pl
from jax.experimental.pallas import tpu as pltpu

_CH = 512
_NBUF = 3


def _body(pos_ref, h_hbm, emb_ref, out_hbm, inbuf, outbuf, rowstage,
          in_sems, out_sems, row_sem):
    b, s, d = h_hbm.shape
    ncpb = s // _CH
    total = b * ncpb
    pos = pos_ref[0]

    def in_cp(g):
        bi, ci = divmod(g, ncpb)
        k = g % _NBUF
        return pltpu.make_async_copy(
            h_hbm.at[bi, pl.ds(ci * _CH, _CH), :], inbuf.at[k], in_sems.at[k])

    def out_cp(g):
        bi, ci = divmod(g, ncpb)
        k = g % _NBUF
        return pltpu.make_async_copy(
            outbuf.at[k], out_hbm.at[bi, pl.ds(ci * _CH, _CH), :],
            out_sems.at[k])

    row_cps = []
    for g in range(min(_NBUF, total)):
        in_cp(g).start()
    for g in range(total):
        bi, ci = divmod(g, ncpb)
        k = g % _NBUF
        in_cp(g).wait()
        if g + 1 >= _NBUF and g + 1 - _NBUF >= 0:
            out_cp(g + 1 - _NBUF).wait()
        if ci == 0:
            outbuf[k, pl.ds(0, 1), :] = emb_ref[pl.ds(pos, 1), :]
        outbuf[k, pl.ds(1, _CH - 1), :] = inbuf[k, pl.ds(0, _CH - 1), :]
        out_cp(g).start()
        if g + 1 < total and (g + 1) % ncpb != 0:
            outbuf[(g + 1) % _NBUF, pl.ds(0, 1), :] = inbuf[k, pl.ds(_CH - 1, 1), :]
        if ci == ncpb - 1:
            rowstage[bi, pl.ds(0, 1), :] = inbuf[k, pl.ds(_CH - 1, 1), :]
            rc = pltpu.make_async_copy(
                rowstage.at[bi], out_hbm.at[bi, pl.ds(s, 1), :], row_sem)
            rc.start()
            row_cps.append(rc)
        nxt = g + _NBUF
        if nxt < total:
            in_cp(nxt).start()
    for g in range(max(0, total - _NBUF + 1), total):
        out_cp(g).wait()
    for rc in row_cps:
        rc.wait()


def kernel(hidden_states, position, embeddings):
    b, s, d = hidden_states.shape
    pos_arr = jnp.asarray(position, jnp.int32).reshape((1,))
    return pl.pallas_call(
        _body,
        out_shape=jax.ShapeDtypeStruct((b, s + 1, d), hidden_states.dtype),
        in_specs=[
            pl.BlockSpec(memory_space=pltpu.SMEM),
            pl.BlockSpec(memory_space=pl.ANY),
            pl.BlockSpec(memory_space=pltpu.VMEM),
        ],
        out_specs=pl.BlockSpec(memory_space=pl.ANY),
        scratch_shapes=[
            pltpu.VMEM((_NBUF, _CH, d), hidden_states.dtype),
            pltpu.VMEM((_NBUF, _CH, d), hidden_states.dtype),
            pltpu.VMEM((b, 1, d), hidden_states.dtype),
            pltpu.SemaphoreType.DMA((_NBUF,)),
            pltpu.SemaphoreType.DMA((_NBUF,)),
            pltpu.SemaphoreType.DMA,
        ],
    )(pos_arr, hidden_states, embeddings)


# speedup vs baseline: 1.0098x; 1.0098x over previous
import jax
import jax.numpy as jnp
from jax.experimental import pallas as pl
from jax.experimental.pallas import tpu as pltpu

_CH = 1024
_NBUF = 3


def _body(pos_ref, h_hbm, emb_ref, out_hbm, inbuf, outbuf, rowstage,
          in_sems, out_sems, row_sem):
    b, s, d = h_hbm.shape
    ncpb = s // _CH
    total = b * ncpb
    pos = pos_ref[0]

    def in_cp(g):
        bi, ci = divmod(g, ncpb)
        k = g % _NBUF
        return pltpu.make_async_copy(
            h_hbm.at[bi, pl.ds(ci * _CH, _CH), :], inbuf.at[k], in_sems.at[k])

    def out_cp(g):
        bi, ci = divmod(g, ncpb)
        k = g % _NBUF
        return pltpu.make_async_copy(
            outbuf.at[k], out_hbm.at[bi, pl.ds(ci * _CH, _CH), :],
            out_sems.at[k])

    row_cps = []
    for g in range(min(_NBUF, total)):
        in_cp(g).start()
    for g in range(total):
        bi, ci = divmod(g, ncpb)
        k = g % _NBUF
        in_cp(g).wait()
        if g + 1 >= _NBUF and g + 1 - _NBUF >= 0:
            out_cp(g + 1 - _NBUF).wait()
        if ci == 0:
            outbuf[k, pl.ds(0, 1), :] = emb_ref[pl.ds(pos, 1), :]
        outbuf[k, pl.ds(1, _CH - 1), :] = inbuf[k, pl.ds(0, _CH - 1), :]
        out_cp(g).start()
        if g + 1 < total and (g + 1) % ncpb != 0:
            outbuf[(g + 1) % _NBUF, pl.ds(0, 1), :] = inbuf[k, pl.ds(_CH - 1, 1), :]
        if ci == ncpb - 1:
            rowstage[bi, pl.ds(0, 1), :] = inbuf[k, pl.ds(_CH - 1, 1), :]
            rc = pltpu.make_async_copy(
                rowstage.at[bi], out_hbm.at[bi, pl.ds(s, 1), :], row_sem)
            rc.start()
            row_cps.append(rc)
        nxt = g + _NBUF
        if nxt < total:
            in_cp(nxt).start()
    for g in range(max(0, total - _NBUF + 1), total):
        out_cp(g).wait()
    for rc in row_cps:
        rc.wait()


def kernel(hidden_states, position, embeddings):
    b, s, d = hidden_states.shape
    pos_arr = jnp.asarray(position, jnp.int32).reshape((1,))
    return pl.pallas_call(
        _body,
        out_shape=jax.ShapeDtypeStruct((b, s + 1, d), hidden_states.dtype),
        in_specs=[
            pl.BlockSpec(memory_space=pltpu.SMEM),
            pl.BlockSpec(memory_space=pl.ANY),
            pl.BlockSpec(memory_space=pltpu.VMEM),
        ],
        out_specs=pl.BlockSpec(memory_space=pl.ANY),
        scratch_shapes=[
            pltpu.VMEM((_NBUF, _CH, d), hidden_states.dtype),
            pltpu.VMEM((_NBUF, _CH, d), hidden_states.dtype),
            pltpu.VMEM((b, 1, d), hidden_states.dtype),
            pltpu.SemaphoreType.DMA((_NBUF,)),
            pltpu.SemaphoreType.DMA((_NBUF,)),
            pltpu.SemaphoreType.DMA,
        ],
    )(pos_arr, hidden_states, embeddings)
